# baseline (speedup 1.0000x reference)
import functools

import jax
import jax.numpy as jnp
from jax.experimental import pallas as pl
from jax.experimental.pallas import tpu as pltpu

_K = 20
_BLK_R = 1024
_NC = 4
_NUM_CLS = 128


def _loss_body(feats_ref, lab_ref, out_ref, fn_ref, *, k, blk_r, nc):
    i = pl.program_id(0)
    b = feats_ref.shape[0]
    w = b // nc

    @pl.when(i == 0)
    def _init():
        x = feats_ref[...]
        nrm = jnp.sqrt(jnp.sum(x * x, axis=1, keepdims=True))
        fn_ref[...] = (x / jnp.maximum(nrm, 1e-12)).astype(jnp.bfloat16)
        fnb = fn_ref[...].astype(jnp.float32)
        cls = jax.lax.broadcasted_iota(jnp.int32, (_NUM_CLS, b), 0)
        onehot = (cls == lab_ref[...]).astype(jnp.bfloat16)
        csum = jax.lax.dot_general(
            onehot,
            fn_ref[...],
            dimension_numbers=(((1,), (0,)), ((), ())),
            preferred_element_type=jnp.float32,
        )
        same_total = jnp.sum(csum * csum, keepdims=True).reshape(1, 1) - jnp.sum(
            fnb * fnb, keepdims=True
        ).reshape(1, 1)
        out_ref[...] = -same_total / b

    f_blk = fn_ref[pl.ds(i * blk_r, blk_r), :]
    rows = i * blk_r + jax.lax.broadcasted_iota(jnp.int32, (blk_r, 1), 0)

    chunks = []
    m = None
    for c in range(nc):
        f_c = fn_ref[pl.ds(c * w, w), :]
        s_c = jax.lax.dot_general(
            f_blk,
            f_c,
            dimension_numbers=(((1,), (1,)), ((), ())),
            preferred_element_type=jnp.float32,
        )
        col_c = c * w + jax.lax.broadcasted_iota(jnp.int32, (blk_r, w), 1)
        v = jnp.where(col_c == rows, -jnp.inf, s_c)
        chunks.append(v)
        m_c = jnp.max(v, axis=1, keepdims=True)
        m = m_c if m is None else jnp.maximum(m, m_c)

    for _ in range(k - 1):
        m_next = None
        for c in range(nc):
            mm = jnp.max(
                jnp.where(chunks[c] < m, chunks[c], -jnp.inf),
                axis=1,
                keepdims=True,
            )
            m_next = mm if m_next is None else jnp.maximum(m_next, mm)
        m = m_next
    t_thr = jnp.maximum(m, -3.4e38)

    sum_gt = jnp.zeros((blk_r, 1), jnp.float32)
    cnt_gt = jnp.zeros((blk_r, 1), jnp.float32)
    for c in range(nc):
        gt = chunks[c] > t_thr
        sum_gt += jnp.sum(jnp.where(gt, chunks[c], 0.0), axis=1, keepdims=True)
        cnt_gt += jnp.sum(gt.astype(jnp.float32), axis=1, keepdims=True)
    take = jnp.maximum(float(k) - cnt_gt, 0.0)
    topk_sum = jnp.sum(sum_gt + take * t_thr, keepdims=True).reshape(1, 1)

    out_ref[...] += topk_sum / b


def kernel(feats, labels):
    b, _ = feats.shape
    lab_col = labels.reshape(1, b)
    grid = b // _BLK_R
    out = pl.pallas_call(
        functools.partial(_loss_body, k=_K, blk_r=_BLK_R, nc=_NC),
        grid=(grid,),
        in_specs=[
            pl.BlockSpec(feats.shape, lambda i: (0, 0)),
            pl.BlockSpec((1, b), lambda i: (0, 0)),
        ],
        out_specs=pl.BlockSpec((1, 1), lambda i: (0, 0)),
        out_shape=jax.ShapeDtypeStruct((1, 1), jnp.float32),
        scratch_shapes=[pltpu.VMEM(feats.shape, jnp.bfloat16)],
    )(feats, lab_col)
    return out[0, 0]


# speedup vs baseline: 1.3035x; 1.3035x over previous
import functools

import jax
import jax.numpy as jnp
from jax.experimental import pallas as pl
from jax.experimental.pallas import tpu as pltpu

_K = 20
_BLK_R = 256
_NC = 4
_NUM_CLS = 128


def _loss_body(feats_ref, lab_ref, out_ref, fn_ref, *, k, blk_r, nc):
    i = pl.program_id(0)
    b = feats_ref.shape[0]
    w = b // nc

    @pl.when(i == 0)
    def _init():
        x = feats_ref[...]
        nrm = jnp.sqrt(jnp.sum(x * x, axis=1, keepdims=True))
        fn_ref[...] = (x / jnp.maximum(nrm, 1e-12)).astype(jnp.bfloat16)
        fnb = fn_ref[...].astype(jnp.float32)
        cls = jax.lax.broadcasted_iota(jnp.int32, (_NUM_CLS, b), 0)
        onehot = (cls == lab_ref[...]).astype(jnp.bfloat16)
        csum = jax.lax.dot_general(
            onehot,
            fn_ref[...],
            dimension_numbers=(((1,), (0,)), ((), ())),
            preferred_element_type=jnp.float32,
        )
        same_total = jnp.sum(csum * csum, keepdims=True).reshape(1, 1) - jnp.sum(
            fnb * fnb, keepdims=True
        ).reshape(1, 1)
        out_ref[...] = -same_total / b

    f_blk = fn_ref[pl.ds(i * blk_r, blk_r), :]
    rows = i * blk_r + jax.lax.broadcasted_iota(jnp.int32, (blk_r, 1), 0)

    chunks = []
    m = None
    for c in range(nc):
        f_c = fn_ref[pl.ds(c * w, w), :]
        s_c = jax.lax.dot_general(
            f_blk,
            f_c,
            dimension_numbers=(((1,), (1,)), ((), ())),
            preferred_element_type=jnp.float32,
        )
        col_c = c * w + jax.lax.broadcasted_iota(jnp.int32, (blk_r, w), 1)
        v = jnp.where(col_c == rows, -jnp.inf, s_c)
        chunks.append(v)
        m_c = jnp.max(v, axis=1, keepdims=True)
        m = m_c if m is None else jnp.maximum(m, m_c)

    for _ in range(k - 1):
        m_next = None
        for c in range(nc):
            mm = jnp.max(
                jnp.where(chunks[c] < m, chunks[c], -jnp.inf),
                axis=1,
                keepdims=True,
            )
            m_next = mm if m_next is None else jnp.maximum(m_next, mm)
        m = m_next
    t_thr = jnp.maximum(m, -3.4e38)

    sum_gt = jnp.zeros((blk_r, 1), jnp.float32)
    cnt_gt = jnp.zeros((blk_r, 1), jnp.float32)
    for c in range(nc):
        gt = chunks[c] > t_thr
        sum_gt += jnp.sum(jnp.where(gt, chunks[c], 0.0), axis=1, keepdims=True)
        cnt_gt += jnp.sum(gt.astype(jnp.float32), axis=1, keepdims=True)
    take = jnp.maximum(float(k) - cnt_gt, 0.0)
    topk_sum = jnp.sum(sum_gt + take * t_thr, keepdims=True).reshape(1, 1)

    out_ref[...] += topk_sum / b


def kernel(feats, labels):
    b, _ = feats.shape
    lab_col = labels.reshape(1, b)
    grid = b // _BLK_R
    out = pl.pallas_call(
        functools.partial(_loss_body, k=_K, blk_r=_BLK_R, nc=_NC),
        grid=(grid,),
        in_specs=[
            pl.BlockSpec(feats.shape, lambda i: (0, 0)),
            pl.BlockSpec((1, b), lambda i: (0, 0)),
        ],
        out_specs=pl.BlockSpec((1, 1), lambda i: (0, 0)),
        out_shape=jax.ShapeDtypeStruct((1, 1), jnp.float32),
        scratch_shapes=[pltpu.VMEM(feats.shape, jnp.bfloat16)],
    )(feats, lab_col)
    return out[0, 0]


# speedup vs baseline: 1.3336x; 1.0231x over previous
import functools

import jax
import jax.numpy as jnp
from jax.experimental import pallas as pl
from jax.experimental.pallas import tpu as pltpu

_K = 20
_BLK_R = 512
_NC = 8
_NUM_CLS = 128


def _loss_body(feats_ref, lab_ref, out_ref, fn_ref, *, k, blk_r, nc):
    i = pl.program_id(0)
    b = feats_ref.shape[0]
    w = b // nc

    @pl.when(i == 0)
    def _init():
        x = feats_ref[...]
        nrm = jnp.sqrt(jnp.sum(x * x, axis=1, keepdims=True))
        fn_ref[...] = (x / jnp.maximum(nrm, 1e-12)).astype(jnp.bfloat16)
        fnb = fn_ref[...].astype(jnp.float32)
        cls = jax.lax.broadcasted_iota(jnp.int32, (_NUM_CLS, b), 0)
        onehot = (cls == lab_ref[...]).astype(jnp.bfloat16)
        csum = jax.lax.dot_general(
            onehot,
            fn_ref[...],
            dimension_numbers=(((1,), (0,)), ((), ())),
            preferred_element_type=jnp.float32,
        )
        same_total = jnp.sum(csum * csum, keepdims=True).reshape(1, 1) - jnp.sum(
            fnb * fnb, keepdims=True
        ).reshape(1, 1)
        out_ref[...] = -same_total / b

    f_blk = fn_ref[pl.ds(i * blk_r, blk_r), :]
    rows = i * blk_r + jax.lax.broadcasted_iota(jnp.int32, (blk_r, 1), 0)

    chunks = []
    m = None
    for c in range(nc):
        f_c = fn_ref[pl.ds(c * w, w), :]
        s_c = jax.lax.dot_general(
            f_blk,
            f_c,
            dimension_numbers=(((1,), (1,)), ((), ())),
            preferred_element_type=jnp.float32,
        )
        col_c = c * w + jax.lax.broadcasted_iota(jnp.int32, (blk_r, w), 1)
        v = jnp.where(col_c == rows, -jnp.inf, s_c)
        chunks.append(v)
        m_c = jnp.max(v, axis=1, keepdims=True)
        m = m_c if m is None else jnp.maximum(m, m_c)

    for _ in range(k - 1):
        m_next = None
        for c in range(nc):
            mm = jnp.max(
                jnp.where(chunks[c] < m, chunks[c], -jnp.inf),
                axis=1,
                keepdims=True,
            )
            m_next = mm if m_next is None else jnp.maximum(m_next, mm)
        m = m_next
    t_thr = jnp.maximum(m, -3.4e38)

    sum_gt = jnp.zeros((blk_r, 1), jnp.float32)
    cnt_gt = jnp.zeros((blk_r, 1), jnp.float32)
    for c in range(nc):
        gt = chunks[c] > t_thr
        sum_gt += jnp.sum(jnp.where(gt, chunks[c], 0.0), axis=1, keepdims=True)
        cnt_gt += jnp.sum(gt.astype(jnp.float32), axis=1, keepdims=True)
    take = jnp.maximum(float(k) - cnt_gt, 0.0)
    topk_sum = jnp.sum(sum_gt + take * t_thr, keepdims=True).reshape(1, 1)

    out_ref[...] += topk_sum / b


def kernel(feats, labels):
    b, _ = feats.shape
    lab_col = labels.reshape(1, b)
    grid = b // _BLK_R
    out = pl.pallas_call(
        functools.partial(_loss_body, k=_K, blk_r=_BLK_R, nc=_NC),
        grid=(grid,),
        in_specs=[
            pl.BlockSpec(feats.shape, lambda i: (0, 0)),
            pl.BlockSpec((1, b), lambda i: (0, 0)),
        ],
        out_specs=pl.BlockSpec((1, 1), lambda i: (0, 0)),
        out_shape=jax.ShapeDtypeStruct((1, 1), jnp.float32),
        scratch_shapes=[pltpu.VMEM(feats.shape, jnp.bfloat16)],
    )(feats, lab_col)
    return out[0, 0]
